# baseline (speedup 1.0000x reference)
import functools
import math

import jax
import jax.numpy as jnp
from jax import lax
from jax.experimental import pallas as pl
from jax.experimental.pallas import tpu as pltpu
from jax.experimental.pallas import tpu_sc as plsc

_D = 64
_SCALE = math.sqrt(_D)
_NW = 32
_BLK = 128
_LANES = 16
_GRPS = _BLK // _LANES


def _emb_body(xp_hbm, xh_hbm, lut2_hbm, out_hbm, pidx_v, offs_v, g_v, t_v,
              sem_in, sem_out):
    n_seq = out_hbm.shape[0]
    wid = lax.axis_index("s") * 2 + lax.axis_index("c")
    bbase = wid * _BLK
    lane = lax.iota(jnp.int32, _LANES)

    pltpu.sync_copy(xp_hbm.at[:, pl.ds(bbase, _BLK)], pidx_v)
    pltpu.sync_copy(xh_hbm.at[:, pl.ds(bbase, _BLK)], offs_v)

    def start_gather(s, buf):
        pltpu.async_copy(
            lut2_hbm.at[pidx_v.at[s]], g_v.at[buf, :, pl.ds(0, 2 * _D)], sem_in
        )

    def wait_gather(buf):
        pltpu.make_async_copy(
            lut2_hbm.at[pl.ds(0, _BLK)], g_v.at[buf, :, pl.ds(0, 2 * _D)], sem_in
        ).wait()

    def start_store(s, buf):
        pltpu.async_copy(
            t_v.at[buf], out_hbm.at[s, :, pl.ds(bbase, _BLK)], sem_out
        )

    def wait_store(buf):
        pltpu.make_async_copy(
            t_v.at[buf], out_hbm.at[0, :, pl.ds(bbase, _BLK)], sem_out
        ).wait()

    def process(s, buf):
        offs = [offs_v[s, pl.ds(g * _LANES, _LANES)] for g in range(_GRPS)]
        rows = [lane + g * _LANES for g in range(_GRPS)]

        @pl.loop(0, _D, unroll=2)
        def _feat(c):
            for g in range(_GRPS):
                v = plsc.load_gather(g_v.at[buf], [rows[g], offs[g] + c])
                t_v[buf, c, pl.ds(g * _LANES, _LANES)] = v * _SCALE

    start_gather(0, 0)

    @pl.loop(0, n_seq, step=2)
    def _seq(s0):
        for b in range(2):
            s = s0 + b
            nxt = 1 - b
            wait_gather(b)

            @pl.when(s + 1 < n_seq)
            def _next_gather():
                start_gather(s + 1, nxt)

            @pl.when(s >= 2)
            def _drain():
                wait_store(b)

            process(s, b)
            start_store(s, b)

    wait_store(0)
    wait_store(1)


def kernel(x, lut):
    b, s = x.shape
    vocab, d = lut.shape
    x_t = x.T
    x_p = lax.shift_right_logical(x_t, 1)
    x_h = lax.shift_left(jnp.bitwise_and(x_t, 1), 6)
    lut2 = lut.reshape(vocab // 2, 2 * d)

    mesh = plsc.VectorSubcoreMesh(core_axis_name="c", subcore_axis_name="s")
    run = functools.partial(
        pl.kernel,
        out_type=jax.ShapeDtypeStruct((s, d, b), jnp.float32),
        mesh=mesh,
        scratch_types=[
            pltpu.VMEM((s, _BLK), jnp.int32),
            pltpu.VMEM((s, _BLK), jnp.int32),
            pltpu.VMEM((2, _BLK, 2 * d + 1), jnp.float32),
            pltpu.VMEM((2, d, _BLK), jnp.float32),
            pltpu.SemaphoreType.DMA,
            pltpu.SemaphoreType.DMA,
        ],
        compiler_params=pltpu.CompilerParams(needs_layout_passes=False),
    )(_emb_body)
    out = run(x_p, x_h, lut2)
    return out.transpose(2, 0, 1)


# speedup vs baseline: 1.2157x; 1.2157x over previous
import functools
import math

import jax
import jax.numpy as jnp
from jax import lax
from jax.experimental import pallas as pl
from jax.experimental.pallas import tpu as pltpu
from jax.experimental.pallas import tpu_sc as plsc

_D = 64
_SCALE = math.sqrt(_D)
_NW = 32
_CHUNK = 128
_LANES = 16
_NBUF = 5
_LEAD = 2


def _emb_body(x_hbm, lut_hbm, out_hbm, idx_v, rows_v, sem_in, sem_out):
    n_chunks = x_hbm.shape[1]
    wid = lax.axis_index("s") * 2 + lax.axis_index("c")
    base = wid * (n_chunks * _CHUNK)

    pltpu.sync_copy(x_hbm.at[wid], idx_v)

    def start_gather(g, b):
        pltpu.async_copy(lut_hbm.at[idx_v.at[g]], rows_v.at[b], sem_in)

    def wait_gather(b):
        pltpu.make_async_copy(
            lut_hbm.at[pl.ds(0, _CHUNK)], rows_v.at[b], sem_in
        ).wait()

    def start_store(g, b):
        pltpu.async_copy(
            rows_v.at[b], out_hbm.at[pl.ds(base + g * _CHUNK, _CHUNK)], sem_out
        )

    def wait_store(b):
        pltpu.make_async_copy(
            rows_v.at[b], out_hbm.at[pl.ds(base, _CHUNK)], sem_out
        ).wait()

    for g in range(_LEAD):
        start_gather(g, g % _NBUF)

    @pl.loop(0, n_chunks, step=_NBUF)
    def _outer(g0):
        for b in range(_NBUF):
            g = g0 + b
            wait_gather(b)

            @pl.loop(0, _CHUNK, unroll=8)
            def _row(i):
                for j in range(_D // _LANES):
                    sl = pl.ds(j * _LANES, _LANES)
                    rows_v[b, i, sl] = rows_v[b, i, sl] * _SCALE

            start_store(g, b)

            h = g + _LEAD
            hb = (b + _LEAD) % _NBUF

            @pl.when(h < n_chunks)
            def _prefetch():
                @pl.when(h >= _NBUF)
                def _drain_prior_store():
                    wait_store(hb)

                start_gather(h, hb)

    for b in range(_NBUF):
        wait_store(b)


def kernel(x, lut):
    b, s = x.shape
    total = b * s
    n_chunks = total // (_NW * _CHUNK)
    x_grid = x.reshape(_NW, n_chunks, _CHUNK)

    mesh = plsc.VectorSubcoreMesh(core_axis_name="c", subcore_axis_name="s")
    run = functools.partial(
        pl.kernel,
        out_type=jax.ShapeDtypeStruct((total, _D), jnp.float32),
        mesh=mesh,
        scratch_types=[
            pltpu.VMEM((n_chunks, _CHUNK), jnp.int32),
            pltpu.VMEM((_NBUF, _CHUNK, _D), jnp.float32),
            pltpu.SemaphoreType.DMA,
            pltpu.SemaphoreType.DMA,
        ],
        compiler_params=pltpu.CompilerParams(use_tc_tiling_on_sc=False),
    )(_emb_body)
    out = run(x_grid, lut)
    return out.reshape(b, s, _D)
